# baseline (speedup 1.0000x reference)
import numpy as np
import jax
import jax.numpy as jnp
from jax.experimental import pallas as pl
from jax.experimental.pallas import tpu as pltpu

BZ = 64
N = 360
T = 100
H = 128
D = 100
HEADS = 2
DH = 64

_WPAD_L = 64768
_RB = 8
_BS = 8


def _fused_kernel(ts_ref, sp_ref, corr_ref,
                  w2_ref, convb_ref, tsfw_ref, tsfb_ref, gatw_ref,
                  asrc_ref, adst_ref, lng_ref, lnb_ref,
                  wpad_ref, awr_ref, bias_ref,
                  out_ref, wf_ref):
    f32 = jnp.float32

    @pl.when(pl.program_id(0) == 0)
    def _build_dense_nonaggr_weights():
        lane = jax.lax.broadcasted_iota(jnp.int32, (1, N), 1)
        for c in range(2):
            def body(k, _):
                rows = []
                for j in range(_RB):
                    i = k * _RB + j
                    off = i * N - (i * (i + 1)) // 2 - i
                    off_al = (off // 128) * 128
                    rem = off - off_al
                    win = wpad_ref[pl.ds(c, 1), pl.ds(off_al, 512)]
                    wrow = pltpu.roll(win, (512 - rem) % 512, axis=1)[:, :N]
                    rows.append(jnp.where(lane > i, wrow, 0.0))
                wf_ref[pl.ds(c * N + k * _RB, _RB), :] = jnp.concatenate(
                    rows, axis=0)
                return 0
            jax.lax.fori_loop(0, N // _RB, body, 0)

    res = []
    for s in range(_BS):
        ts2 = ts_ref[s]
        h_pre = jax.lax.dot_general(ts2, w2_ref[...],
                                    (((0,), (0,)), ((), ())),
                                    preferred_element_type=f32)
        h = jnp.maximum(h_pre + convb_ref[...], 0.0)
        nf = (jnp.dot(h, tsfw_ref[...], preferred_element_type=f32)
              + tsfb_ref[...])
        hh = jnp.dot(nf, gatw_ref[...], preferred_element_type=f32)
        a_src = jnp.dot(hh, asrc_ref[...], preferred_element_type=f32)
        a_dst_t = jax.lax.dot_general(adst_ref[...], hh,
                                      (((0,), (1,)), ((), ())),
                                      preferred_element_type=f32)

        adj = sp_ref[s] != 0.0
        ones_col = jnp.ones((N, 1), f32)
        outs = []
        for hd in range(HEADS):
            e = a_src[:, hd:hd + 1] + a_dst_t[hd:hd + 1, :]
            e = jnp.maximum(e, 0.2 * e)
            e = jnp.where(adj, e, -60.0)
            p = jnp.exp(e)
            hh_ext = jnp.concatenate([hh[:, hd * DH:(hd + 1) * DH], ones_col],
                                     axis=1)
            acc = jnp.dot(p, hh_ext, preferred_element_type=f32)
            inv_s = 1.0 / acc[:, DH:DH + 1]
            outs.append(acc[:, :DH] * inv_s)
        go = jnp.concatenate(outs, axis=1)
        mu = jnp.mean(go, axis=1, keepdims=True)
        var = jnp.mean((go - mu) ** 2, axis=1, keepdims=True)
        y = (go - mu) * jax.lax.rsqrt(var + 1e-5) * lng_ref[...] + lnb_ref[...]

        corr = corr_ref[s]
        ones_row = jnp.ones((1, N), f32)
        rv0 = jax.lax.dot_general(ones_row, corr * wf_ref[:N, :],
                                  (((1,), (0,)), ((), ())),
                                  preferred_element_type=f32)
        rv1 = jax.lax.dot_general(ones_row, corr * wf_ref[N:, :],
                                  (((1,), (0,)), ((), ())),
                                  preferred_element_type=f32)
        ry0 = jax.lax.dot_general(ones_row, y * awr_ref[0],
                                  (((1,), (0,)), ((), ())),
                                  preferred_element_type=f32)
        ry1 = jax.lax.dot_general(ones_row, y * awr_ref[1],
                                  (((1,), (0,)), ((), ())),
                                  preferred_element_type=f32)
        v0 = jnp.sum(rv0) + jnp.sum(ry0)
        v1 = jnp.sum(rv1) + jnp.sum(ry1)
        res.append(jnp.concatenate([v0.reshape(1, 1, 1), v1.reshape(1, 1, 1)],
                                   axis=2))
    out_ref[...] = jnp.concatenate(res, axis=0) + bias_ref[...]


@jax.jit
def kernel(timeseries, sparse_connection, corr, conv_w, conv_b, ts_fc_w,
           ts_fc_b, gat_w, gat_a_src, gat_a_dst, ln_g, ln_b,
           nonaggr_w, nonaggr_b, aggr_w, aggr_b):
    w2 = conv_w[:, 0, 0:2].T
    R = N * (N - 1) // 2
    wpad = jnp.concatenate(
        [jnp.zeros((2, 1), jnp.float32), nonaggr_w.T,
         jnp.zeros((2, _WPAD_L - R - 1), jnp.float32)], axis=1)
    r_idx = np.arange(HEADS * DH)
    c_idx = np.repeat(np.arange(HEADS), DH)
    a_src_m = jnp.zeros((HEADS * DH, HEADS), jnp.float32).at[r_idx, c_idx].set(
        gat_a_src.reshape(-1))
    a_dst_m = jnp.zeros((HEADS * DH, HEADS), jnp.float32).at[r_idx, c_idx].set(
        gat_a_dst.reshape(-1))
    awr = jnp.transpose(aggr_w.reshape(N, H, 2), (2, 0, 1))
    bias = (nonaggr_b + aggr_b).reshape(1, 1, 2)
    ts2 = jax.lax.slice(timeseries, (0, T - 2, 0), (BZ, T, N))

    const = lambda shape: pl.BlockSpec(shape, lambda b: (0,) * len(shape))
    in_specs = [
            pl.BlockSpec((_BS, 2, N), lambda b: (b, 0, 0)),
            pl.BlockSpec((_BS, N, N), lambda b: (b, 0, 0)),
            pl.BlockSpec((_BS, N, N), lambda b: (b, 0, 0)),
            const((2, H)), const((1, H)), const((H, D)), const((1, D)),
            const((D, H)), const((H, HEADS)), const((H, HEADS)),
            const((1, H)), const((1, H)),
            const((2, _WPAD_L)), const((2, N, H)), const((1, 1, 2)),
    ]
    out = pl.pallas_call(
        _fused_kernel,
        grid=(BZ // _BS,),
        in_specs=in_specs,
        out_specs=pl.BlockSpec((_BS, 1, 2), lambda b: (b, 0, 0)),
        out_shape=jax.ShapeDtypeStruct((BZ, 1, 2), jnp.float32),
        scratch_shapes=[pltpu.VMEM((2 * N, N), jnp.float32)],
    )(ts2, sparse_connection, corr,
      w2, conv_b.reshape(1, H), ts_fc_w, ts_fc_b.reshape(1, D), gat_w,
      a_src_m, a_dst_m, ln_g.reshape(1, H), ln_b.reshape(1, H),
      wpad, awr, bias)
    return out.reshape(BZ, 2)


# speedup vs baseline: 1.2043x; 1.2043x over previous
import numpy as np
import jax
import jax.numpy as jnp
from jax.experimental import pallas as pl
from jax.experimental.pallas import tpu as pltpu

BZ = 64
N = 360
T = 100
H = 128
D = 100
HEADS = 2
DH = 64

_WPAD_L = 64768
_RB = 8
_BS = 8


def _fused_kernel(ts_ref, sp_ref, corr_ref,
                  w2_ref, convb_ref, tsfw_ref, tsfb_ref, gatw_ref,
                  asrc_ref, adst_ref, lng_ref, lnb_ref,
                  wpad_ref, awr_ref, bias_ref,
                  out_ref, wf_ref):
    f32 = jnp.float32

    @pl.when(pl.program_id(0) == 0)
    def _build_dense_nonaggr_weights():
        lane = jax.lax.broadcasted_iota(jnp.int32, (1, N), 1)
        for c in range(2):
            def body(k, _):
                rows = []
                for j in range(_RB):
                    i = k * _RB + j
                    off = i * N - (i * (i + 1)) // 2 - i
                    off_al = (off // 128) * 128
                    rem = off - off_al
                    win = wpad_ref[pl.ds(c, 1), pl.ds(off_al, 512)]
                    wrow = pltpu.roll(win, (512 - rem) % 512, axis=1)[:, :N]
                    rows.append(jnp.where(lane > i, wrow, 0.0))
                wf_ref[pl.ds(c * N + k * _RB, _RB), :] = jnp.concatenate(
                    rows, axis=0)
                return 0
            jax.lax.fori_loop(0, N // _RB, body, 0)

    res = []
    for s in range(_BS):
        ts2 = ts_ref[s]
        h_pre = jax.lax.dot_general(ts2, w2_ref[...],
                                    (((0,), (0,)), ((), ())),
                                    preferred_element_type=f32)
        h = jnp.maximum(h_pre + convb_ref[...], 0.0)
        nf = (jnp.dot(h, tsfw_ref[...], preferred_element_type=f32)
              + tsfb_ref[...])
        hh = jnp.dot(nf, gatw_ref[...], preferred_element_type=f32)
        a_src = jnp.dot(hh, asrc_ref[...], preferred_element_type=f32)
        a_dst_t = jax.lax.dot_general(adst_ref[...], hh,
                                      (((0,), (1,)), ((), ())),
                                      preferred_element_type=f32)

        adj = sp_ref[s] != 0.0
        ones_col = jnp.ones((N, 1), f32)
        outs = []
        for hd in range(HEADS):
            e = a_src[:, hd:hd + 1] + a_dst_t[hd:hd + 1, :]
            e = jnp.maximum(e, 0.2 * e)
            e = jnp.where(adj, e, -60.0)
            p = jnp.exp(e)
            hh_ext = jnp.concatenate([hh[:, hd * DH:(hd + 1) * DH], ones_col],
                                     axis=1)
            acc = jnp.dot(p, hh_ext, preferred_element_type=f32)
            inv_s = 1.0 / acc[:, DH:DH + 1]
            outs.append(acc[:, :DH] * inv_s)
        go = jnp.concatenate(outs, axis=1)
        mu = jnp.mean(go, axis=1, keepdims=True)
        var = jnp.mean((go - mu) ** 2, axis=1, keepdims=True)
        y = (go - mu) * jax.lax.rsqrt(var + 1e-5) * lng_ref[...] + lnb_ref[...]

        corr = corr_ref[s]
        ones_row = jnp.ones((1, N), f32)
        rv0 = jax.lax.dot_general(ones_row, corr * wf_ref[:N, :],
                                  (((1,), (0,)), ((), ())),
                                  preferred_element_type=f32)
        rv1 = jax.lax.dot_general(ones_row, corr * wf_ref[N:, :],
                                  (((1,), (0,)), ((), ())),
                                  preferred_element_type=f32)
        v0 = jnp.sum(rv0) + jnp.sum(y * awr_ref[0])
        v1 = jnp.sum(rv1) + jnp.sum(y * awr_ref[1])
        res.append(jnp.concatenate([v0.reshape(1, 1, 1), v1.reshape(1, 1, 1)],
                                   axis=2))
    out_ref[...] = jnp.concatenate(res, axis=0) + bias_ref[...]


@jax.jit
def kernel(timeseries, sparse_connection, corr, conv_w, conv_b, ts_fc_w,
           ts_fc_b, gat_w, gat_a_src, gat_a_dst, ln_g, ln_b,
           nonaggr_w, nonaggr_b, aggr_w, aggr_b):
    w2 = conv_w[:, 0, 0:2].T
    R = N * (N - 1) // 2
    wpad = jnp.concatenate(
        [jnp.zeros((2, 1), jnp.float32), nonaggr_w.T,
         jnp.zeros((2, _WPAD_L - R - 1), jnp.float32)], axis=1)
    r_idx = np.arange(HEADS * DH)
    c_idx = np.repeat(np.arange(HEADS), DH)
    a_src_m = jnp.zeros((HEADS * DH, HEADS), jnp.float32).at[r_idx, c_idx].set(
        gat_a_src.reshape(-1))
    a_dst_m = jnp.zeros((HEADS * DH, HEADS), jnp.float32).at[r_idx, c_idx].set(
        gat_a_dst.reshape(-1))
    awr = jnp.transpose(aggr_w.reshape(N, H, 2), (2, 0, 1))
    bias = (nonaggr_b + aggr_b).reshape(1, 1, 2)
    ts2 = jax.lax.slice(timeseries, (0, T - 2, 0), (BZ, T, N))

    const = lambda shape: pl.BlockSpec(shape, lambda b: (0,) * len(shape))
    in_specs = [
            pl.BlockSpec((_BS, 2, N), lambda b: (b, 0, 0)),
            pl.BlockSpec((_BS, N, N), lambda b: (b, 0, 0)),
            pl.BlockSpec((_BS, N, N), lambda b: (b, 0, 0)),
            const((2, H)), const((1, H)), const((H, D)), const((1, D)),
            const((D, H)), const((H, HEADS)), const((H, HEADS)),
            const((1, H)), const((1, H)),
            const((2, _WPAD_L)), const((2, N, H)), const((1, 1, 2)),
    ]
    out = pl.pallas_call(
        _fused_kernel,
        grid=(BZ // _BS,),
        in_specs=in_specs,
        out_specs=pl.BlockSpec((_BS, 1, 2), lambda b: (b, 0, 0)),
        out_shape=jax.ShapeDtypeStruct((BZ, 1, 2), jnp.float32),
        scratch_shapes=[pltpu.VMEM((2 * N, N), jnp.float32)],
    )(ts2, sparse_connection, corr,
      w2, conv_b.reshape(1, H), ts_fc_w, ts_fc_b.reshape(1, D), gat_w,
      a_src_m, a_dst_m, ln_g.reshape(1, H), ln_b.reshape(1, H),
      wpad, awr, bias)
    return out.reshape(BZ, 2)
